# baseline (speedup 1.0000x reference)
import functools

import jax
import jax.numpy as jnp
from jax import lax
from jax.experimental import pallas as pl
from jax.experimental.pallas import tpu as pltpu
from jax.experimental.pallas import tpu_sc as plsc

_NC = 2
_NS = 16
_K = 125
_G = 16
_CH = 80
_BN = 1000

_F32 = jnp.float32


def _dot(a, b):
    return lax.dot_general(a, b, (((a.ndim - 1,), (0,)), ((), ())),
                           precision=lax.Precision.HIGHEST,
                           preferred_element_type=_F32)


@functools.lru_cache(maxsize=None)
def _sc_agg_fn(n, d, e_rows):
    rw = e_rows // (_NC * _NS)
    nchunk = n // _CH
    ntrip = (nchunk + _NS - 1) // _NS
    mesh = plsc.VectorSubcoreMesh(core_axis_name="c", subcore_axis_name="s")

    out_type = jax.ShapeDtypeStruct((_NC * n, d), _F32)
    scratch = [
        pltpu.VMEM((_G, _K), jnp.int32),
        pltpu.VMEM((_G, _K), jnp.int32),
        pltpu.VMEM((_K, d), _F32),
        pltpu.VMEM((_K, d), _F32),
        pltpu.VMEM_SHARED((n, d), _F32),
        pltpu.SemaphoreType.DMA,
        pltpu.SemaphoreType.DMA,
    ]

    @functools.partial(pl.kernel, out_type=out_type, mesh=mesh,
                       scratch_types=scratch)
    def agg(table_h, src_h, dst_h, z_h, out_h, srcv, dstv, rows0,
            rows1, acc, sem0, sem1):
        c = lax.axis_index("c")
        s = lax.axis_index("s")
        w = c * _NS + s

        @pl.loop(0, ntrip)
        def _(t):
            cid = s + t * _NS

            @pl.when(cid < nchunk)
            def _():
                off = cid * _CH
                pltpu.sync_copy(z_h.at[pl.ds(off, _CH)],
                                acc.at[pl.ds(off, _CH)])

        plsc.subcore_barrier()

        @pl.loop(0, rw // _G)
        def _(g):
            pltpu.sync_copy(src_h.at[pl.ds(w * rw + g * _G, _G)], srcv)
            pltpu.sync_copy(dst_h.at[pl.ds(w * rw + g * _G, _G)], dstv)
            pltpu.async_copy(table_h.at[srcv.at[0]], rows0, sem0)

            @pl.loop(0, _G // 2)
            def _(p):
                j0 = 2 * p
                pltpu.make_async_copy(
                    table_h.at[srcv.at[j0]], rows0, sem0).wait()
                pltpu.async_copy(table_h.at[srcv.at[j0 + 1]], rows1, sem1)
                pltpu.sync_copy(rows0, acc.at[dstv.at[j0]], add=True)
                pltpu.make_async_copy(
                    table_h.at[srcv.at[j0 + 1]], rows1, sem1).wait()

                @pl.when(j0 + 2 < _G)
                def _():
                    pltpu.async_copy(
                        table_h.at[srcv.at[j0 + 2]], rows0, sem0)

                pltpu.sync_copy(rows1, acc.at[dstv.at[j0 + 1]], add=True)

        plsc.subcore_barrier()

        @pl.loop(0, ntrip)
        def _(t):
            cid = s + t * _NS

            @pl.when(cid < nchunk)
            def _():
                off = cid * _CH
                pltpu.sync_copy(acc.at[pl.ds(off, _CH)],
                                out_h.at[pl.ds(c * n + off, _CH)])

    return agg


@functools.lru_cache(maxsize=None)
def _sc_deg_fn(n, d, e_rows):
    rw = e_rows // (_NC * _NS)
    nchunk = n // _CH
    ntrip = (nchunk + _NS - 1) // _NS
    mesh = plsc.VectorSubcoreMesh(core_axis_name="c", subcore_axis_name="s")

    out_type = jax.ShapeDtypeStruct((_NC * n, d), _F32)
    scratch = [
        pltpu.VMEM((_G, _K), jnp.int32),
        pltpu.VMEM((_K, d), _F32),
        pltpu.VMEM_SHARED((n, d), _F32),
    ]

    @functools.partial(pl.kernel, out_type=out_type, mesh=mesh,
                       scratch_types=scratch)
    def deg(dst_h, z_h, ones_h, out_h, dstv, onesv, acc):
        c = lax.axis_index("c")
        s = lax.axis_index("s")
        w = c * _NS + s
        pltpu.sync_copy(ones_h, onesv)

        @pl.loop(0, ntrip)
        def _(t):
            cid = s + t * _NS

            @pl.when(cid < nchunk)
            def _():
                off = cid * _CH
                pltpu.sync_copy(z_h.at[pl.ds(off, _CH)],
                                acc.at[pl.ds(off, _CH)])

        plsc.subcore_barrier()

        @pl.loop(0, rw // _G)
        def _(g):
            pltpu.sync_copy(dst_h.at[pl.ds(w * rw + g * _G, _G)], dstv)

            @pl.loop(0, _G)
            def _(j):
                pltpu.sync_copy(onesv, acc.at[dstv.at[j]], add=True)

        plsc.subcore_barrier()

        @pl.loop(0, ntrip)
        def _(t):
            cid = s + t * _NS

            @pl.when(cid < nchunk)
            def _():
                off = cid * _CH
                pltpu.sync_copy(acc.at[pl.ds(off, _CH)],
                                out_h.at[pl.ds(c * n + off, _CH)])

    return deg


def _hca_body(h_ref, q_ref, r_ref, o_ref):
    q = q_ref[...]
    r = r_ref[0:1, :]
    hs = (h_ref[0], h_ref[1], h_ref[2])
    e = [jnp.sum(jnp.tanh(_dot(h, q)) * r, axis=1, keepdims=True) for h in hs]
    m = jnp.maximum(jnp.maximum(e[0], e[1]), e[2])
    x = [jnp.exp(ei - m) for ei in e]
    ssum = x[0] + x[1] + x[2]
    o_ref[...] = (x[0] * hs[0] + x[1] * hs[1] + x[2] * hs[2]) / ssum


def _hca(hiddens, q, r_pad):
    n, d = hiddens.shape[1], hiddens.shape[2]
    return pl.pallas_call(
        _hca_body,
        grid=(n // _BN,),
        in_specs=[
            pl.BlockSpec((3, _BN, d), lambda i: (0, i, 0)),
            pl.BlockSpec((d, d), lambda i: (0, 0)),
            pl.BlockSpec((8, d), lambda i: (0, 0)),
        ],
        out_specs=pl.BlockSpec((_BN, d), lambda i: (i, 0)),
        out_shape=jax.ShapeDtypeStruct((n, d), _F32),
    )(hiddens, q, r_pad)


def _layer_body(p_ref, d_ref, w_ref, o_ref):
    p = p_ref[0] + p_ref[1]
    deg = jnp.maximum(d_ref[0, :, 0:1] + d_ref[1, :, 0:1], 1.0)
    o_ref[...] = jnp.maximum(_dot(p / deg, w_ref[...]), 0.0)


def _layer(parts, degparts, wmat):
    n, d = parts.shape[1], parts.shape[2]
    return pl.pallas_call(
        _layer_body,
        grid=(n // _BN,),
        in_specs=[
            pl.BlockSpec((_NC, _BN, d), lambda i: (0, i, 0)),
            pl.BlockSpec((_NC, _BN, d), lambda i: (0, i, 0)),
            pl.BlockSpec((d, d), lambda i: (0, 0)),
        ],
        out_specs=pl.BlockSpec((_BN, d), lambda i: (i, 0)),
        out_shape=jax.ShapeDtypeStruct((n, d), _F32),
    )(parts, degparts, wmat)


def _final_body(p_ref, d_ref, s_ref, w2, wz, uz, wr, ur, wh, uh, o_ref):
    p = p_ref[0] + p_ref[1]
    deg = jnp.maximum(d_ref[0, :, 0:1] + d_ref[1, :, 0:1], 1.0)
    cur = jnp.maximum(_dot(p / deg, w2[...]), 0.0)
    sh = s_ref[...]
    z = jax.nn.sigmoid(_dot(cur, wz[...]) + _dot(sh, uz[...]))
    r = jax.nn.sigmoid(_dot(cur, wr[...]) + _dot(sh, ur[...]))
    ht = jnp.tanh(_dot(cur, wh[...]) + _dot(r * sh, uh[...]))
    h = (1.0 - z) * sh + z * ht
    o_ref[...] = jnp.maximum(h, 0.0)


def _final(parts, degparts, short, W2, Wz, Uz, Wr, Ur, Wh, Uh):
    n, d = parts.shape[1], parts.shape[2]
    wspec = pl.BlockSpec((d, d), lambda i: (0, 0))
    return pl.pallas_call(
        _final_body,
        grid=(n // _BN,),
        in_specs=[
            pl.BlockSpec((_NC, _BN, d), lambda i: (0, i, 0)),
            pl.BlockSpec((_NC, _BN, d), lambda i: (0, i, 0)),
            pl.BlockSpec((_BN, d), lambda i: (i, 0)),
        ] + [wspec] * 7,
        out_specs=pl.BlockSpec((_BN, d), lambda i: (i, 0)),
        out_shape=jax.ShapeDtypeStruct((n, d), _F32),
    )(parts, degparts, short, W2, Wz, Uz, Wr, Ur, Wh, Uh)


def kernel(edge_index, prev, hiddens, W1, W2, Q, r_att, Wz, Uz, Wr, Ur, Wh, Uh):
    n, d = prev.shape
    e = edge_index.shape[1]
    assert e % _K == 0 and (e // _K) % (_NC * _NS) == 0 and n % _CH == 0

    e_rows = e // _K
    src2 = edge_index[0].reshape(e_rows, _K).astype(jnp.int32)
    dst2 = edge_index[1].reshape(e_rows, _K).astype(jnp.int32)
    zeros = jnp.zeros((n, d), _F32)
    ones = jnp.ones((_K, d), _F32)
    r_pad = jnp.zeros((8, d), _F32).at[0].set(r_att.astype(_F32))

    prev = prev.astype(_F32)

    parts1 = _sc_agg_fn(n, d, e_rows)(
        prev, src2, dst2, zeros).reshape(_NC, n, d)
    degparts = _sc_deg_fn(n, d, e_rows)(
        dst2, zeros, ones).reshape(_NC, n, d)

    short = _hca(hiddens.astype(_F32), Q, r_pad)
    h1 = _layer(parts1, degparts, W1)

    parts2 = _sc_agg_fn(n, d, e_rows)(
        h1, src2, dst2, zeros).reshape(_NC, n, d)

    return _final(parts2, degparts, short, W2, Wz, Uz, Wr, Ur, Wh, Uh)


# speedup vs baseline: 1.1734x; 1.1734x over previous
import functools

import jax
import jax.numpy as jnp
from jax import lax
from jax.experimental import pallas as pl
from jax.experimental.pallas import tpu as pltpu
from jax.experimental.pallas import tpu_sc as plsc

_NC = 2
_NS = 16
_K = 125
_G = 16
_CH = 80
_BN = 1000

_F32 = jnp.float32


def _dot(a, b):
    return lax.dot_general(a, b, (((a.ndim - 1,), (0,)), ((), ())),
                           precision=lax.Precision.DEFAULT,
                           preferred_element_type=_F32)


@functools.lru_cache(maxsize=None)
def _sc_agg_fn(n, d, e_rows):
    rw = e_rows // (_NC * _NS)
    nchunk = n // _CH
    ntrip = (nchunk + _NS - 1) // _NS
    mesh = plsc.VectorSubcoreMesh(core_axis_name="c", subcore_axis_name="s")

    out_type = jax.ShapeDtypeStruct((_NC * n, d), _F32)
    scratch = [
        pltpu.VMEM((_G, _K), jnp.int32),
        pltpu.VMEM((_G, _K), jnp.int32),
        pltpu.VMEM((_K, d), _F32),
        pltpu.VMEM((_K, d), _F32),
        pltpu.VMEM_SHARED((n, d), _F32),
        pltpu.SemaphoreType.DMA,
        pltpu.SemaphoreType.DMA,
    ]

    @functools.partial(pl.kernel, out_type=out_type, mesh=mesh,
                       scratch_types=scratch)
    def agg(table_h, src_h, dst_h, z_h, out_h, srcv, dstv, rows0,
            rows1, acc, sem0, sem1):
        c = lax.axis_index("c")
        s = lax.axis_index("s")
        w = c * _NS + s

        @pl.loop(0, ntrip)
        def _(t):
            cid = s + t * _NS

            @pl.when(cid < nchunk)
            def _():
                off = cid * _CH
                pltpu.sync_copy(z_h.at[pl.ds(off, _CH)],
                                acc.at[pl.ds(off, _CH)])

        plsc.subcore_barrier()

        @pl.loop(0, rw // _G)
        def _(g):
            pltpu.sync_copy(src_h.at[pl.ds(w * rw + g * _G, _G)], srcv)
            pltpu.sync_copy(dst_h.at[pl.ds(w * rw + g * _G, _G)], dstv)
            pltpu.async_copy(table_h.at[srcv.at[0]], rows0, sem0)

            @pl.loop(0, _G // 2)
            def _(p):
                j0 = 2 * p
                pltpu.make_async_copy(
                    table_h.at[srcv.at[j0]], rows0, sem0).wait()
                pltpu.async_copy(table_h.at[srcv.at[j0 + 1]], rows1, sem1)
                pltpu.sync_copy(rows0, acc.at[dstv.at[j0]], add=True)
                pltpu.make_async_copy(
                    table_h.at[srcv.at[j0 + 1]], rows1, sem1).wait()

                @pl.when(j0 + 2 < _G)
                def _():
                    pltpu.async_copy(
                        table_h.at[srcv.at[j0 + 2]], rows0, sem0)

                pltpu.sync_copy(rows1, acc.at[dstv.at[j0 + 1]], add=True)

        plsc.subcore_barrier()

        @pl.loop(0, ntrip)
        def _(t):
            cid = s + t * _NS

            @pl.when(cid < nchunk)
            def _():
                off = cid * _CH
                pltpu.sync_copy(acc.at[pl.ds(off, _CH)],
                                out_h.at[pl.ds(c * n + off, _CH)])

    return agg


@functools.lru_cache(maxsize=None)
def _sc_deg_fn(n, d, e_rows):
    rw = e_rows // (_NC * _NS)
    nchunk = n // _CH
    ntrip = (nchunk + _NS - 1) // _NS
    mesh = plsc.VectorSubcoreMesh(core_axis_name="c", subcore_axis_name="s")

    out_type = jax.ShapeDtypeStruct((_NC * n, d), _F32)
    scratch = [
        pltpu.VMEM((_G, _K), jnp.int32),
        pltpu.VMEM((_K, d), _F32),
        pltpu.VMEM_SHARED((n, d), _F32),
    ]

    @functools.partial(pl.kernel, out_type=out_type, mesh=mesh,
                       scratch_types=scratch)
    def deg(dst_h, z_h, ones_h, out_h, dstv, onesv, acc):
        c = lax.axis_index("c")
        s = lax.axis_index("s")
        w = c * _NS + s
        pltpu.sync_copy(ones_h, onesv)

        @pl.loop(0, ntrip)
        def _(t):
            cid = s + t * _NS

            @pl.when(cid < nchunk)
            def _():
                off = cid * _CH
                pltpu.sync_copy(z_h.at[pl.ds(off, _CH)],
                                acc.at[pl.ds(off, _CH)])

        plsc.subcore_barrier()

        @pl.loop(0, rw // _G)
        def _(g):
            pltpu.sync_copy(dst_h.at[pl.ds(w * rw + g * _G, _G)], dstv)

            @pl.loop(0, _G)
            def _(j):
                pltpu.sync_copy(onesv, acc.at[dstv.at[j]], add=True)

        plsc.subcore_barrier()

        @pl.loop(0, ntrip)
        def _(t):
            cid = s + t * _NS

            @pl.when(cid < nchunk)
            def _():
                off = cid * _CH
                pltpu.sync_copy(acc.at[pl.ds(off, _CH)],
                                out_h.at[pl.ds(c * n + off, _CH)])

    return deg


def _hca_body(h_ref, q_ref, r_ref, o_ref):
    q = q_ref[...]
    r = r_ref[0:1, :]
    hs = (h_ref[0], h_ref[1], h_ref[2])
    e = [jnp.sum(jnp.tanh(_dot(h, q)) * r, axis=1, keepdims=True) for h in hs]
    m = jnp.maximum(jnp.maximum(e[0], e[1]), e[2])
    x = [jnp.exp(ei - m) for ei in e]
    ssum = x[0] + x[1] + x[2]
    o_ref[...] = (x[0] * hs[0] + x[1] * hs[1] + x[2] * hs[2]) / ssum


def _hca(hiddens, q, r_pad):
    n, d = hiddens.shape[1], hiddens.shape[2]
    return pl.pallas_call(
        _hca_body,
        grid=(n // _BN,),
        in_specs=[
            pl.BlockSpec((3, _BN, d), lambda i: (0, i, 0)),
            pl.BlockSpec((d, d), lambda i: (0, 0)),
            pl.BlockSpec((8, d), lambda i: (0, 0)),
        ],
        out_specs=pl.BlockSpec((_BN, d), lambda i: (i, 0)),
        out_shape=jax.ShapeDtypeStruct((n, d), _F32),
    )(hiddens, q, r_pad)


def _layer_body(p_ref, d_ref, w_ref, o_ref):
    p = p_ref[0] + p_ref[1]
    deg = jnp.maximum(d_ref[0, :, 0:1] + d_ref[1, :, 0:1], 1.0)
    o_ref[...] = jnp.maximum(_dot(p / deg, w_ref[...]), 0.0)


def _layer(parts, degparts, wmat):
    n, d = parts.shape[1], parts.shape[2]
    return pl.pallas_call(
        _layer_body,
        grid=(n // _BN,),
        in_specs=[
            pl.BlockSpec((_NC, _BN, d), lambda i: (0, i, 0)),
            pl.BlockSpec((_NC, _BN, d), lambda i: (0, i, 0)),
            pl.BlockSpec((d, d), lambda i: (0, 0)),
        ],
        out_specs=pl.BlockSpec((_BN, d), lambda i: (i, 0)),
        out_shape=jax.ShapeDtypeStruct((n, d), _F32),
    )(parts, degparts, wmat)


def _final_body(p_ref, d_ref, s_ref, w2, wz, uz, wr, ur, wh, uh, o_ref):
    p = p_ref[0] + p_ref[1]
    deg = jnp.maximum(d_ref[0, :, 0:1] + d_ref[1, :, 0:1], 1.0)
    cur = jnp.maximum(_dot(p / deg, w2[...]), 0.0)
    sh = s_ref[...]
    z = jax.nn.sigmoid(_dot(cur, wz[...]) + _dot(sh, uz[...]))
    r = jax.nn.sigmoid(_dot(cur, wr[...]) + _dot(sh, ur[...]))
    ht = jnp.tanh(_dot(cur, wh[...]) + _dot(r * sh, uh[...]))
    h = (1.0 - z) * sh + z * ht
    o_ref[...] = jnp.maximum(h, 0.0)


def _final(parts, degparts, short, W2, Wz, Uz, Wr, Ur, Wh, Uh):
    n, d = parts.shape[1], parts.shape[2]
    wspec = pl.BlockSpec((d, d), lambda i: (0, 0))
    return pl.pallas_call(
        _final_body,
        grid=(n // _BN,),
        in_specs=[
            pl.BlockSpec((_NC, _BN, d), lambda i: (0, i, 0)),
            pl.BlockSpec((_NC, _BN, d), lambda i: (0, i, 0)),
            pl.BlockSpec((_BN, d), lambda i: (i, 0)),
        ] + [wspec] * 7,
        out_specs=pl.BlockSpec((_BN, d), lambda i: (i, 0)),
        out_shape=jax.ShapeDtypeStruct((n, d), _F32),
    )(parts, degparts, short, W2, Wz, Uz, Wr, Ur, Wh, Uh)


def kernel(edge_index, prev, hiddens, W1, W2, Q, r_att, Wz, Uz, Wr, Ur, Wh, Uh):
    n, d = prev.shape
    e = edge_index.shape[1]
    assert e % _K == 0 and (e // _K) % (_NC * _NS) == 0 and n % _CH == 0

    e_rows = e // _K
    src2 = edge_index[0].reshape(e_rows, _K).astype(jnp.int32)
    dst2 = edge_index[1].reshape(e_rows, _K).astype(jnp.int32)
    zeros = jnp.zeros((n, d), _F32)
    ones = jnp.ones((_K, d), _F32)
    r_pad = jnp.zeros((8, d), _F32).at[0].set(r_att.astype(_F32))

    prev = prev.astype(_F32)

    parts1 = _sc_agg_fn(n, d, e_rows)(
        prev, src2, dst2, zeros).reshape(_NC, n, d)
    degparts = _sc_deg_fn(n, d, e_rows)(
        dst2, zeros, ones).reshape(_NC, n, d)

    short = _hca(hiddens.astype(_F32), Q, r_pad)
    h1 = _layer(parts1, degparts, W1)

    parts2 = _sc_agg_fn(n, d, e_rows)(
        h1, src2, dst2, zeros).reshape(_NC, n, d)

    return _final(parts2, degparts, short, W2, Wz, Uz, Wr, Ur, Wh, Uh)


# speedup vs baseline: 1.2043x; 1.0263x over previous
import functools

import jax
import jax.numpy as jnp
from jax import lax
from jax.experimental import pallas as pl
from jax.experimental.pallas import tpu as pltpu
from jax.experimental.pallas import tpu_sc as plsc

_NC = 2
_NS = 16
_K = 125
_G = 16
_CH = 80
_BN = 1000

_F32 = jnp.float32


def _dot(a, b):
    return lax.dot_general(a, b, (((a.ndim - 1,), (0,)), ((), ())),
                           precision=lax.Precision.DEFAULT,
                           preferred_element_type=_F32)


@functools.lru_cache(maxsize=None)
def _sc_agg_fn(n, d, e_rows):
    rw = e_rows // (_NC * _NS)
    nchunk = n // _CH
    ntrip = (nchunk + _NS - 1) // _NS
    mesh = plsc.VectorSubcoreMesh(core_axis_name="c", subcore_axis_name="s")

    out_type = jax.ShapeDtypeStruct((_NC * n, d), _F32)
    scratch = [
        pltpu.VMEM((_G, _K), jnp.int32),
        pltpu.VMEM((_G, _K), jnp.int32),
        pltpu.VMEM((_K, d), _F32),
        pltpu.VMEM((_K, d), _F32),
        pltpu.VMEM_SHARED((n, d), _F32),
        pltpu.SemaphoreType.DMA,
        pltpu.SemaphoreType.DMA,
    ]

    @functools.partial(pl.kernel, out_type=out_type, mesh=mesh,
                       scratch_types=scratch)
    def agg(table_h, e_h, z_h, out_h, srcv, dstv, rows0,
            rows1, acc, sem0, sem1):
        c = lax.axis_index("c")
        s = lax.axis_index("s")
        w = c * _NS + s

        @pl.loop(0, ntrip)
        def _(t):
            cid = s + t * _NS

            @pl.when(cid < nchunk)
            def _():
                off = cid * _CH
                pltpu.sync_copy(z_h.at[pl.ds(off, _CH)],
                                acc.at[pl.ds(off, _CH)])

        plsc.subcore_barrier()

        @pl.loop(0, rw // _G)
        def _(g):
            pltpu.sync_copy(e_h.at[0].at[pl.ds(w * rw + g * _G, _G)], srcv)
            pltpu.sync_copy(e_h.at[1].at[pl.ds(w * rw + g * _G, _G)], dstv)
            pltpu.async_copy(table_h.at[srcv.at[0]], rows0, sem0)

            @pl.loop(0, _G // 2)
            def _(p):
                j0 = 2 * p
                pltpu.make_async_copy(
                    table_h.at[srcv.at[j0]], rows0, sem0).wait()
                pltpu.async_copy(table_h.at[srcv.at[j0 + 1]], rows1, sem1)
                pltpu.sync_copy(rows0, acc.at[dstv.at[j0]], add=True)
                pltpu.make_async_copy(
                    table_h.at[srcv.at[j0 + 1]], rows1, sem1).wait()

                @pl.when(j0 + 2 < _G)
                def _():
                    pltpu.async_copy(
                        table_h.at[srcv.at[j0 + 2]], rows0, sem0)

                pltpu.sync_copy(rows1, acc.at[dstv.at[j0 + 1]], add=True)

        plsc.subcore_barrier()

        @pl.loop(0, ntrip)
        def _(t):
            cid = s + t * _NS

            @pl.when(cid < nchunk)
            def _():
                off = cid * _CH
                pltpu.sync_copy(acc.at[pl.ds(off, _CH)],
                                out_h.at[pl.ds(c * n + off, _CH)])

    return agg


@functools.lru_cache(maxsize=None)
def _sc_deg_fn(n, d, e_rows):
    rw = e_rows // (_NC * _NS)
    nchunk = n // _CH
    ntrip = (nchunk + _NS - 1) // _NS
    mesh = plsc.VectorSubcoreMesh(core_axis_name="c", subcore_axis_name="s")

    out_type = jax.ShapeDtypeStruct((_NC * n, d), _F32)
    scratch = [
        pltpu.VMEM((_G, _K), jnp.int32),
        pltpu.VMEM((_K, d), _F32),
        pltpu.VMEM_SHARED((n, d), _F32),
    ]

    @functools.partial(pl.kernel, out_type=out_type, mesh=mesh,
                       scratch_types=scratch)
    def deg(e_h, z_h, ones_h, out_h, dstv, onesv, acc):
        c = lax.axis_index("c")
        s = lax.axis_index("s")
        w = c * _NS + s
        pltpu.sync_copy(ones_h, onesv)

        @pl.loop(0, ntrip)
        def _(t):
            cid = s + t * _NS

            @pl.when(cid < nchunk)
            def _():
                off = cid * _CH
                pltpu.sync_copy(z_h.at[pl.ds(off, _CH)],
                                acc.at[pl.ds(off, _CH)])

        plsc.subcore_barrier()

        @pl.loop(0, rw // _G)
        def _(g):
            pltpu.sync_copy(e_h.at[1].at[pl.ds(w * rw + g * _G, _G)], dstv)

            @pl.loop(0, _G)
            def _(j):
                pltpu.sync_copy(onesv, acc.at[dstv.at[j]], add=True)

        plsc.subcore_barrier()

        @pl.loop(0, ntrip)
        def _(t):
            cid = s + t * _NS

            @pl.when(cid < nchunk)
            def _():
                off = cid * _CH
                pltpu.sync_copy(acc.at[pl.ds(off, _CH)],
                                out_h.at[pl.ds(c * n + off, _CH)])

    return deg


def _hca_body(h_ref, q_ref, r_ref, o_ref):
    q = q_ref[...]
    r = r_ref[0:1, :]
    hs = (h_ref[0], h_ref[1], h_ref[2])
    e = [jnp.sum(jnp.tanh(_dot(h, q)) * r, axis=1, keepdims=True) for h in hs]
    m = jnp.maximum(jnp.maximum(e[0], e[1]), e[2])
    x = [jnp.exp(ei - m) for ei in e]
    ssum = x[0] + x[1] + x[2]
    o_ref[...] = (x[0] * hs[0] + x[1] * hs[1] + x[2] * hs[2]) / ssum


def _hca(hiddens, q, r_pad):
    n, d = hiddens.shape[1], hiddens.shape[2]
    return pl.pallas_call(
        _hca_body,
        grid=(n // _BN,),
        in_specs=[
            pl.BlockSpec((3, _BN, d), lambda i: (0, i, 0)),
            pl.BlockSpec((d, d), lambda i: (0, 0)),
            pl.BlockSpec((8, d), lambda i: (0, 0)),
        ],
        out_specs=pl.BlockSpec((_BN, d), lambda i: (i, 0)),
        out_shape=jax.ShapeDtypeStruct((n, d), _F32),
    )(hiddens, q, r_pad)


def _layer_body(p_ref, d_ref, w_ref, o_ref):
    p = p_ref[0] + p_ref[1]
    deg = jnp.maximum(d_ref[0, :, 0:1] + d_ref[1, :, 0:1], 1.0)
    o_ref[...] = jnp.maximum(_dot(p / deg, w_ref[...]), 0.0)


def _layer(parts, degparts, wmat):
    n, d = parts.shape[1], parts.shape[2]
    return pl.pallas_call(
        _layer_body,
        grid=(n // _BN,),
        in_specs=[
            pl.BlockSpec((_NC, _BN, d), lambda i: (0, i, 0)),
            pl.BlockSpec((_NC, _BN, d), lambda i: (0, i, 0)),
            pl.BlockSpec((d, d), lambda i: (0, 0)),
        ],
        out_specs=pl.BlockSpec((_BN, d), lambda i: (i, 0)),
        out_shape=jax.ShapeDtypeStruct((n, d), _F32),
    )(parts, degparts, wmat)


def _final_body(p_ref, d_ref, s_ref, w2, wz, uz, wr, ur, wh, uh, o_ref):
    p = p_ref[0] + p_ref[1]
    deg = jnp.maximum(d_ref[0, :, 0:1] + d_ref[1, :, 0:1], 1.0)
    cur = jnp.maximum(_dot(p / deg, w2[...]), 0.0)
    sh = s_ref[...]
    z = jax.nn.sigmoid(_dot(cur, wz[...]) + _dot(sh, uz[...]))
    r = jax.nn.sigmoid(_dot(cur, wr[...]) + _dot(sh, ur[...]))
    ht = jnp.tanh(_dot(cur, wh[...]) + _dot(r * sh, uh[...]))
    h = (1.0 - z) * sh + z * ht
    o_ref[...] = jnp.maximum(h, 0.0)


def _final(parts, degparts, short, W2, Wz, Uz, Wr, Ur, Wh, Uh):
    n, d = parts.shape[1], parts.shape[2]
    wspec = pl.BlockSpec((d, d), lambda i: (0, 0))
    return pl.pallas_call(
        _final_body,
        grid=(n // _BN,),
        in_specs=[
            pl.BlockSpec((_NC, _BN, d), lambda i: (0, i, 0)),
            pl.BlockSpec((_NC, _BN, d), lambda i: (0, i, 0)),
            pl.BlockSpec((_BN, d), lambda i: (i, 0)),
        ] + [wspec] * 7,
        out_specs=pl.BlockSpec((_BN, d), lambda i: (i, 0)),
        out_shape=jax.ShapeDtypeStruct((n, d), _F32),
    )(parts, degparts, short, W2, Wz, Uz, Wr, Ur, Wh, Uh)


def kernel(edge_index, prev, hiddens, W1, W2, Q, r_att, Wz, Uz, Wr, Ur, Wh, Uh):
    n, d = prev.shape
    e = edge_index.shape[1]
    assert e % _K == 0 and (e // _K) % (_NC * _NS) == 0 and n % _CH == 0

    e_rows = e // _K
    edges3 = edge_index.reshape(2, e_rows, _K).astype(jnp.int32)
    zeros = jnp.zeros((n, d), _F32)
    ones = jnp.ones((_K, d), _F32)
    r_pad = jnp.zeros((8, d), _F32).at[0].set(r_att.astype(_F32))

    prev = prev.astype(_F32)

    parts1 = _sc_agg_fn(n, d, e_rows)(
        prev, edges3, zeros).reshape(_NC, n, d)
    degparts = _sc_deg_fn(n, d, e_rows)(
        edges3, zeros, ones).reshape(_NC, n, d)

    short = _hca(hiddens.astype(_F32), Q, r_pad)
    h1 = _layer(parts1, degparts, W1)

    parts2 = _sc_agg_fn(n, d, e_rows)(
        h1, edges3, zeros).reshape(_NC, n, d)

    return _final(parts2, degparts, short, W2, Wz, Uz, Wr, Ur, Wh, Uh)
